# Initial kernel scaffold; baseline (speedup 1.0000x reference)
import jax
import jax.numpy as jnp
from jax.experimental import pallas as pl


def kernel(graph, feature, node_id, edge_id):
    raise NotImplementedError("write your pallas kernel here")


# speedup vs baseline: 1.8866x; 1.8866x over previous
import functools

import jax
import jax.numpy as jnp
from jax import lax
from jax.experimental import pallas as pl
from jax.experimental.pallas import tpu as pltpu
from jax.experimental.pallas import tpu_sc as plsc

N_NODES = 50000
D_FEAT = 256
N_GRAPHS = 512

NUM_CORES = 2
NUM_SUBCORES = 16
NUM_WORKERS = NUM_CORES * NUM_SUBCORES
SEG_PER_W = N_GRAPHS // NUM_WORKERS
LANES = 16
CHUNK = 128

_mesh = plsc.VectorSubcoreMesh(core_axis_name="c", subcore_axis_name="s")


@functools.partial(
    pl.kernel,
    out_type=jax.ShapeDtypeStruct((N_GRAPHS, D_FEAT), jnp.float32),
    mesh=_mesh,
    scratch_types=[
        pltpu.VMEM((32,), jnp.int32),
        pltpu.VMEM((CHUNK,), jnp.int32),
        pltpu.VMEM((CHUNK, D_FEAT), jnp.float32),
        pltpu.VMEM((SEG_PER_W + 1, D_FEAT), jnp.float32),
    ],
)
def _pool(feat_hbm, nid_hbm, bnd_hbm, out_hbm, bnd_v, ids_v, rows_v, acc_v):
    wid = lax.axis_index("s") * NUM_CORES + lax.axis_index("c")
    seg_lo = pl.multiple_of(wid * SEG_PER_W, SEG_PER_W)

    pltpu.sync_copy(bnd_hbm.at[pl.ds(seg_lo, 32)], bnd_v)
    blo = bnd_v[pl.ds(0, LANES)]
    bhi = bnd_v[pl.ds(LANES, LANES)]
    row_lo = blo[0]
    row_hi = bhi[0]

    zeros = jnp.zeros((LANES,), jnp.float32)

    def zero_body(s, carry):
        for f in range(D_FEAT // LANES):
            acc_v[s, pl.ds(f * LANES, LANES)] = zeros
        return carry

    lax.fori_loop(0, SEG_PER_W + 1, zero_body, 0)

    start = (row_lo >> 3) << 3
    nchunks = (row_hi - start + CHUNK - 1) >> 7

    def chunk_body(k, carry):
        u = start + k * CHUNK
        o = pl.multiple_of(jnp.minimum(u, N_NODES - CHUNK), 8)
        pltpu.sync_copy(feat_hbm.at[pl.ds(o, CHUNK)], rows_v)
        pltpu.sync_copy(nid_hbm.at[pl.ds(o, CHUNK)], ids_v)
        lo_r = jnp.maximum(u, row_lo)

        def grp_body(j, c2):
            base = j * LANES
            iv = ids_v[pl.ds(base, LANES)]
            for t in range(LANES):
                r = o + base + t
                seg = iv[t]
                valid = (r >= lo_r) & (r < row_hi)
                lseg = jnp.where(valid, seg - seg_lo, SEG_PER_W)
                for f in range(D_FEAT // LANES):
                    sl = pl.ds(f * LANES, LANES)
                    plsc.addupdate(acc_v.at[lseg, sl], rows_v[base + t, sl])
            return c2

        lax.fori_loop(0, CHUNK // LANES, grp_body, 0)
        return carry

    lax.fori_loop(0, nchunks, chunk_body, 0)

    pltpu.sync_copy(
        acc_v.at[pl.ds(0, SEG_PER_W)], out_hbm.at[pl.ds(seg_lo, SEG_PER_W)]
    )


def kernel(graph, feature, node_id, edge_id):
    bnd = jnp.searchsorted(
        node_id, jnp.arange(N_GRAPHS + 1, dtype=jnp.int32), side="left"
    ).astype(jnp.int32)
    bnd = jnp.concatenate(
        [bnd, jnp.full((31,), N_NODES, jnp.int32)]
    )
    return _pool(feature, node_id, bnd)


# speedup vs baseline: 3.2244x; 1.7091x over previous
import functools

import jax
import jax.numpy as jnp
from jax import lax
from jax.experimental import pallas as pl
from jax.experimental.pallas import tpu as pltpu
from jax.experimental.pallas import tpu_sc as plsc

N_NODES = 50000
D_FEAT = 256
N_GRAPHS = 512

NUM_CORES = 2
NUM_SUBCORES = 16
NUM_WORKERS = NUM_CORES * NUM_SUBCORES
SEG_PER_W = N_GRAPHS // NUM_WORKERS
LANES = 16
CHUNK = 64

_mesh = plsc.VectorSubcoreMesh(core_axis_name="c", subcore_axis_name="s")


@functools.partial(
    pl.kernel,
    out_type=jax.ShapeDtypeStruct((N_GRAPHS, D_FEAT), jnp.float32),
    mesh=_mesh,
    scratch_types=[
        pltpu.VMEM((N_NODES + LANES,), jnp.int32),
        pltpu.VMEM((CHUNK, D_FEAT), jnp.float32),
        pltpu.VMEM((CHUNK, D_FEAT), jnp.float32),
        pltpu.VMEM((SEG_PER_W + 1, D_FEAT), jnp.float32),
        pltpu.SemaphoreType.DMA,
        pltpu.SemaphoreType.DMA,
    ],
)
def _pool(feat_hbm, nid_hbm, out_hbm, nid_v, rows0, rows1, acc_v, sem0, sem1):
    wid = lax.axis_index("s") * NUM_CORES + lax.axis_index("c")
    seg_lo = pl.multiple_of(wid * SEG_PER_W, SEG_PER_W)

    nid_cp = pltpu.async_copy(nid_hbm, nid_v.at[pl.ds(0, N_NODES)], sem0)
    nid_v[pl.ds(N_NODES, LANES)] = jnp.full((LANES,), N_GRAPHS, jnp.int32)

    zeros = jnp.zeros((LANES,), jnp.float32)

    def zero_body(s, carry):
        for f in range(D_FEAT // LANES):
            acc_v[s, pl.ds(f * LANES, LANES)] = zeros
        return carry

    lax.fori_loop(0, SEG_PER_W + 1, zero_body, 0)
    nid_cp.wait()

    def bsearch(target):
        def search_body(_, carry):
            lo_c, hi_c = carry
            mid = (lo_c + hi_c) >> 1
            val = nid_v[pl.ds(mid, LANES)][0]
            less = val < target
            return (
                jnp.where(less, mid + 1, lo_c),
                jnp.where(less, hi_c, mid),
            )

        lo_f, _ = lax.fori_loop(0, 17, search_body, (0, N_NODES))
        return lo_f

    row_lo = bsearch(seg_lo)
    row_hi = bsearch(seg_lo + SEG_PER_W)

    start = (row_lo >> 3) << 3
    nchunks = (row_hi - start + CHUNK - 1) // CHUNK

    def chunk_off(k):
        u = start + k * CHUNK
        return u, pl.multiple_of(jnp.minimum(u, N_NODES - CHUNK), 8)

    def dma_start(k, buf, sem):
        _, o = chunk_off(k)
        pltpu.async_copy(feat_hbm.at[pl.ds(o, CHUNK)], buf, sem)

    def dma_wait(buf, sem):
        pltpu.make_async_copy(feat_hbm.at[pl.ds(0, CHUNK)], buf, sem).wait()

    def compute(k, buf):
        u, o = chunk_off(k)
        lo_r = jnp.maximum(u, row_lo)

        def grp_body(j, c2):
            base = j * LANES
            iv = nid_v[pl.ds(o + base, LANES)]
            rvec = (o + base) + lax.iota(jnp.int32, LANES)
            valid = (rvec >= lo_r) & (rvec < row_hi)
            lseg_vec = jnp.where(valid, iv - seg_lo, SEG_PER_W)
            for t in range(LANES):
                lseg = lseg_vec[t]
                for f in range(D_FEAT // LANES):
                    sl = pl.ds(f * LANES, LANES)
                    plsc.addupdate(acc_v.at[lseg, sl], buf[base + t, sl])
            return c2

        lax.fori_loop(0, CHUNK // LANES, grp_body, 0)

    @pl.when(nchunks > 0)
    def _():
        dma_start(0, rows0, sem0)

    def pair_body(p, carry):
        k0 = 2 * p
        k1 = k0 + 1

        @pl.when(k1 < nchunks)
        def _():
            dma_start(k1, rows1, sem1)

        dma_wait(rows0, sem0)
        compute(k0, rows0)

        @pl.when(k1 < nchunks)
        def _():
            @pl.when(k1 + 1 < nchunks)
            def _():
                dma_start(k1 + 1, rows0, sem0)

            dma_wait(rows1, sem1)
            compute(k1, rows1)

        return carry

    lax.fori_loop(0, (nchunks + 1) // 2, pair_body, 0)

    pltpu.sync_copy(
        acc_v.at[pl.ds(0, SEG_PER_W)], out_hbm.at[pl.ds(seg_lo, SEG_PER_W)]
    )


def kernel(graph, feature, node_id, edge_id):
    return _pool(feature, node_id)


# speedup vs baseline: 6.2971x; 1.9530x over previous
import functools

import jax
import jax.numpy as jnp
from jax import lax
from jax.experimental import pallas as pl
from jax.experimental.pallas import tpu as pltpu
from jax.experimental.pallas import tpu_sc as plsc

N_NODES = 50000
D_FEAT = 256
N_GRAPHS = 512

NUM_CORES = 2
NUM_SUBCORES = 16
NUM_WORKERS = NUM_CORES * NUM_SUBCORES
SEG_PER_W = N_GRAPHS // NUM_WORKERS
LANES = 16
NBLK = D_FEAT // LANES
CHUNK = 64

_mesh = plsc.VectorSubcoreMesh(core_axis_name="c", subcore_axis_name="s")


@functools.partial(
    pl.kernel,
    out_type=jax.ShapeDtypeStruct((N_GRAPHS, D_FEAT), jnp.float32),
    mesh=_mesh,
    scratch_types=[
        pltpu.VMEM((N_NODES + LANES,), jnp.int32),
        pltpu.VMEM((CHUNK, D_FEAT), jnp.float32),
        pltpu.VMEM((CHUNK, D_FEAT), jnp.float32),
        pltpu.VMEM(((SEG_PER_W + 1) * D_FEAT,), jnp.float32),
        pltpu.SemaphoreType.DMA,
        pltpu.SemaphoreType.DMA,
    ],
)
def _pool(feat_hbm, nid_hbm, out_hbm, nid_v, rows0, rows1, acc_v, sem0, sem1):
    wid = lax.axis_index("s") * NUM_CORES + lax.axis_index("c")
    seg_lo = pl.multiple_of(wid * SEG_PER_W, SEG_PER_W)

    nid_cp = pltpu.async_copy(nid_hbm, nid_v.at[pl.ds(0, N_NODES)], sem0)
    nid_v[pl.ds(N_NODES, LANES)] = jnp.full((LANES,), N_GRAPHS, jnp.int32)

    zeros = jnp.zeros((LANES,), jnp.float32)

    def zero_body(s, carry):
        acc_v[pl.ds(s * LANES, LANES)] = zeros
        return carry

    lax.fori_loop(0, (SEG_PER_W + 1) * NBLK, zero_body, 0)
    nid_cp.wait()

    def bsearch(target):
        def search_body(_, carry):
            lo_c, hi_c = carry
            mid = (lo_c + hi_c) >> 1
            val = nid_v[pl.ds(mid, LANES)][0]
            less = val < target
            return (
                jnp.where(less, mid + 1, lo_c),
                jnp.where(less, hi_c, mid),
            )

        lo_f, _ = lax.fori_loop(0, 17, search_body, (0, N_NODES))
        return lo_f

    row_lo = bsearch(seg_lo)
    row_hi = bsearch(seg_lo + SEG_PER_W)

    start = (row_lo >> 3) << 3
    npairs = (row_hi - start + 2 * CHUNK - 1) // (2 * CHUNK)

    def chunk_off(k):
        u = start + k * CHUNK
        return u, pl.multiple_of(jnp.minimum(u, N_NODES - CHUNK), 8)

    def dma_start(k, buf, sem):
        _, o = chunk_off(k)
        pltpu.async_copy(feat_hbm.at[pl.ds(o, CHUNK)], buf, sem)

    def dma_wait(buf, sem):
        pltpu.make_async_copy(feat_hbm.at[pl.ds(0, CHUNK)], buf, sem).wait()

    def compute(k, buf, cur, regs):
        u, o = chunk_off(k)
        lo_r = jnp.maximum(u, row_lo)

        def grp_body(j, carry):
            cur_c, regs_c = carry
            base = j * LANES
            iv = nid_v[pl.ds(o + base, LANES)]
            lseg_v = iv - seg_lo
            for t in range(LANES):
                r = o + base + t
                valid = (r >= lo_r) & (r < row_hi)
                nxt = jnp.where(valid, lseg_v[t], cur_c)
                changed = nxt != cur_c

                @pl.when(changed)
                def _(cur_s=cur_c, regs_s=regs_c):
                    for f in range(NBLK):
                        plsc.addupdate(
                            acc_v.at[pl.ds(cur_s * D_FEAT + f * LANES, LANES)],
                            regs_s[f],
                        )

                regs_c = tuple(
                    jnp.where(changed, zeros, regs_c[f])
                    + jnp.where(valid, buf[base + t, pl.ds(f * LANES, LANES)], zeros)
                    for f in range(NBLK)
                )
                cur_c = nxt
            return cur_c, regs_c

        return lax.fori_loop(0, CHUNK // LANES, grp_body, (cur, regs))

    @pl.when(npairs > 0)
    def _():
        dma_start(0, rows0, sem0)

    def pair_body(p, carry):
        cur_c, regs_c = carry
        k0 = 2 * p
        dma_start(k0 + 1, rows1, sem1)
        dma_wait(rows0, sem0)
        cur_c, regs_c = compute(k0, rows0, cur_c, regs_c)

        @pl.when(p + 1 < npairs)
        def _():
            dma_start(k0 + 2, rows0, sem0)

        dma_wait(rows1, sem1)
        cur_c, regs_c = compute(k0 + 1, rows1, cur_c, regs_c)
        return cur_c, regs_c

    cur0 = jnp.int32(SEG_PER_W)
    regs0 = tuple(zeros for _ in range(NBLK))
    cur_f, regs_f = lax.fori_loop(0, npairs, pair_body, (cur0, regs0))

    for f in range(NBLK):
        plsc.addupdate(
            acc_v.at[pl.ds(cur_f * D_FEAT + f * LANES, LANES)], regs_f[f]
        )

    for s in range(SEG_PER_W):
        pltpu.sync_copy(
            acc_v.at[pl.ds(s * D_FEAT, D_FEAT)], out_hbm.at[seg_lo + s]
        )


def kernel(graph, feature, node_id, edge_id):
    return _pool(feature, node_id)
